# Initial kernel scaffold; baseline (speedup 1.0000x reference)
import jax
import jax.numpy as jnp
from jax.experimental import pallas as pl


def kernel(X, edge_values, W, edge_index):
    raise NotImplementedError("write your pallas kernel here")


# speedup vs baseline: 4.4268x; 4.4268x over previous
import functools

import jax
import jax.numpy as jnp
from jax import lax
from jax.experimental import pallas as pl
from jax.experimental.pallas import tpu as pltpu
from jax.experimental.pallas import tpu_sc as plsc

N = 10000
D = 128
E = 320000
NC = 2
NS = 16
NW = NC * NS
EPW = E // NW
C = 80
NCHUNK = EPW // C
ZR = 200
NZCHUNK = N // ZR
ZPS = -(-NZCHUNK // NS)


def _sc_body(x_hbm, src_hbm, dst_hbm, ev_hbm, out_hbm,
             src_v, dst_v, ev_v, rows_v, stage_v, agg_sh, sem):
  cid = lax.axis_index("c")
  sid = lax.axis_index("s")
  wid = cid * NS + sid

  def zero_row(r, _):
    for dpart in range(D // 16):
      stage_v[r, pl.ds(dpart * 16, 16)] = jnp.zeros((16,), jnp.float32)
    return 0
  lax.fori_loop(0, ZR, zero_row, 0)
  for z in range(ZPS):
    t = sid + z * NS
    @pl.when(t < NZCHUNK)
    def _():
      pltpu.sync_copy(stage_v, agg_sh.at[pl.ds(t * ZR, ZR)])
  plsc.subcore_barrier()

  def chunk_body(i, _):
    base = wid * EPW + i * C
    pltpu.sync_copy(src_hbm.at[pl.ds(base, C)], src_v)
    pltpu.sync_copy(dst_hbm.at[pl.ds(base, C)], dst_v)
    pltpu.sync_copy(ev_hbm.at[pl.ds(base, C)], ev_v)
    pltpu.async_copy(x_hbm.at[src_v], rows_v, sem).wait()

    def scale_group(g, _):
      ev16 = ev_v[pl.ds(g * 16, 16)]
      for j in range(16):
        e = g * 16 + j
        evs = lax.gather(
            ev16, jnp.full((16, 1), j, jnp.int32),
            lax.GatherDimensionNumbers(
                offset_dims=(), collapsed_slice_dims=(0,),
                start_index_map=(0,)),
            (1,), mode=lax.GatherScatterMode.PROMISE_IN_BOUNDS)
        for dpart in range(D // 16):
          sl = pl.ds(dpart * 16, 16)
          rows_v[e, sl] = rows_v[e, sl] * evs
      return 0
    lax.fori_loop(0, C // 16, scale_group, 0)

    pltpu.sync_copy(rows_v, agg_sh.at[dst_v], add=True)
    return 0
  lax.fori_loop(0, NCHUNK, chunk_body, 0)
  plsc.subcore_barrier()

  for z in range(ZPS):
    t = sid + z * NS
    @pl.when(t < NZCHUNK)
    def _():
      pltpu.sync_copy(agg_sh.at[pl.ds(t * ZR, ZR)], stage_v)
      pltpu.sync_copy(stage_v, out_hbm.at[cid, pl.ds(t * ZR, ZR)])


_sc_agg = pl.kernel(
    _sc_body,
    out_type=jax.ShapeDtypeStruct((NC, N, D), jnp.float32),
    mesh=plsc.VectorSubcoreMesh(
        core_axis_name="c", subcore_axis_name="s",
        num_cores=NC, num_subcores=NS),
    scratch_types=[
        pltpu.VMEM((C,), jnp.int32),
        pltpu.VMEM((C,), jnp.int32),
        pltpu.VMEM((C,), jnp.float32),
        pltpu.VMEM((C, D), jnp.float32),
        pltpu.VMEM((ZR, D), jnp.float32),
        pltpu.VMEM_SHARED((N, D), jnp.float32),
        pltpu.SemaphoreType.DMA,
    ],
)


def _tc_body(p0_ref, p1_ref, x_ref, w_ref, o_ref):
  acc = p0_ref[...] + p1_ref[...] + x_ref[...]
  y = jnp.dot(acc, w_ref[...], preferred_element_type=jnp.float32)
  o_ref[...] = jnp.where(y >= 0, y, 0.2 * y)


_BR = 1000


def _tc_proj(p0, p1, X, W):
  return pl.pallas_call(
      _tc_body,
      grid=(N // _BR,),
      in_specs=[
          pl.BlockSpec((_BR, D), lambda i: (i, 0)),
          pl.BlockSpec((_BR, D), lambda i: (i, 0)),
          pl.BlockSpec((_BR, D), lambda i: (i, 0)),
          pl.BlockSpec((D, D), lambda i: (0, 0)),
      ],
      out_specs=pl.BlockSpec((_BR, D), lambda i: (i, 0)),
      out_shape=jax.ShapeDtypeStruct((N, D), jnp.float32),
  )(p0, p1, X, W)


@jax.jit
def kernel(X, edge_values, W, edge_index):
  dst = edge_index[0]
  src = edge_index[1]
  partials = _sc_agg(X, src, dst, edge_values)
  return _tc_proj(partials[0], partials[1], X, W)


# speedup vs baseline: 9.9475x; 2.2471x over previous
import functools

import jax
import jax.numpy as jnp
from jax import lax
from jax.experimental import pallas as pl
from jax.experimental.pallas import tpu as pltpu
from jax.experimental.pallas import tpu_sc as plsc

N = 10000
D = 128
E = 320000
NC = 2
NS = 16
NW = NC * NS
EPW = E // NW
C = 80
NCHUNK = EPW // C
ZR = C
NZCHUNK = N // ZR
ZPS = -(-NZCHUNK // NS)
NTRI = -(-NCHUNK // 3)


def _sc_body(x_hbm, ei_hbm, ev_hbm, out_hbm,
             eb0, eb1, eb2, ev0, ev1, ev2, rows0, rows1, rows2, agg_sh,
             isem0, isem1, isem2, gsem0, gsem1, gsem2,
             ssem0, ssem1, ssem2):
  cid = lax.axis_index("c")
  sid = lax.axis_index("s")
  wid = cid * NS + sid
  eb = (eb0, eb1, eb2)
  evb = (ev0, ev1, ev2)
  rows = (rows0, rows1, rows2)
  isem = (isem0, isem1, isem2)
  gsem = (gsem0, gsem1, gsem2)
  ssem = (ssem0, ssem1, ssem2)

  pltpu.async_copy(ei_hbm.at[wid, 0], eb[0], isem[0])
  pltpu.async_copy(ev_hbm.at[wid, 0], evb[0], isem[0])
  pltpu.async_copy(ei_hbm.at[wid, 1], eb[1], isem[1])
  pltpu.async_copy(ev_hbm.at[wid, 1], evb[1], isem[1])
  pltpu.make_async_copy(ei_hbm.at[wid, 0], eb[0], isem[0]).wait()
  pltpu.make_async_copy(ev_hbm.at[wid, 0], evb[0], isem[0]).wait()
  pltpu.async_copy(x_hbm.at[eb[0].at[0]], rows[0], gsem[0])

  def zero_row(r, _):
    for dpart in range(D // 16):
      rows2[r, pl.ds(dpart * 16, 16)] = jnp.zeros((16,), jnp.float32)
    return 0
  lax.fori_loop(0, ZR, zero_row, 0)
  for z in range(ZPS):
    t = sid + z * NS
    @pl.when(t < NZCHUNK)
    def _():
      pltpu.sync_copy(rows2, agg_sh.at[pl.ds(t * ZR, ZR)])
  plsc.subcore_barrier()

  def scale_rows(rb, evv):
    def scale_group(g, _):
      ev16 = evv[pl.ds(g * 16, 16)]
      for j in range(16):
        e = g * 16 + j
        evs = lax.gather(
            ev16, jnp.full((16, 1), j, jnp.int32),
            lax.GatherDimensionNumbers(
                offset_dims=(), collapsed_slice_dims=(0,),
                start_index_map=(0,)),
            (1,), mode=lax.GatherScatterMode.PROMISE_IN_BOUNDS)
        for dpart in range(D // 16):
          sl = pl.ds(dpart * 16, 16)
          rb[e, sl] = rb[e, sl] * evs
      return 0
    lax.fori_loop(0, C // 16, scale_group, 0)

  def tri_body(p, _):
    for b in range(3):
      nb = (b + 1) % 3
      pb = (b + 2) % 3
      c = p * 3 + b
      @pl.when(c < NCHUNK)
      def _():
        @pl.when(c >= 1)
        def _():
          pltpu.make_async_copy(
              rows[pb], agg_sh.at[eb[pb].at[1]], ssem[pb]).wait()
        @pl.when(c + 2 < NCHUNK)
        def _():
          pltpu.async_copy(ei_hbm.at[wid, c + 2], eb[pb], isem[pb])
          pltpu.async_copy(ev_hbm.at[wid, c + 2], evb[pb], isem[pb])
        @pl.when(c + 1 < NCHUNK)
        def _():
          pltpu.make_async_copy(
              ei_hbm.at[wid, c + 1], eb[nb], isem[nb]).wait()
          pltpu.make_async_copy(
              ev_hbm.at[wid, c + 1], evb[nb], isem[nb]).wait()
          pltpu.async_copy(x_hbm.at[eb[nb].at[0]], rows[nb], gsem[nb])
        pltpu.make_async_copy(
            x_hbm.at[eb[b].at[0]], rows[b], gsem[b]).wait()
        scale_rows(rows[b], evb[b])
        pltpu.async_copy(rows[b], agg_sh.at[eb[b].at[1]], ssem[b],
                         add=True)
    return 0
  lax.fori_loop(0, NTRI, tri_body, 0)
  lastb = (NCHUNK - 1) % 3
  pltpu.make_async_copy(
      rows[lastb], agg_sh.at[eb[lastb].at[1]], ssem[lastb]).wait()
  plsc.subcore_barrier()

  for z in range(ZPS):
    t = sid + z * NS
    @pl.when(t < NZCHUNK)
    def _():
      pltpu.sync_copy(agg_sh.at[pl.ds(t * ZR, ZR)], rows2)
      pltpu.sync_copy(rows2, out_hbm.at[cid, pl.ds(t * ZR, ZR)])


_sc_agg = pl.kernel(
    _sc_body,
    out_type=jax.ShapeDtypeStruct((NC, N, D), jnp.float32),
    mesh=plsc.VectorSubcoreMesh(
        core_axis_name="c", subcore_axis_name="s",
        num_cores=NC, num_subcores=NS),
    scratch_types=[
        pltpu.VMEM((2, C), jnp.int32),
        pltpu.VMEM((2, C), jnp.int32),
        pltpu.VMEM((2, C), jnp.int32),
        pltpu.VMEM((C,), jnp.float32),
        pltpu.VMEM((C,), jnp.float32),
        pltpu.VMEM((C,), jnp.float32),
        pltpu.VMEM((C, D), jnp.float32),
        pltpu.VMEM((C, D), jnp.float32),
        pltpu.VMEM((C, D), jnp.float32),
        pltpu.VMEM_SHARED((N, D), jnp.float32),
        pltpu.SemaphoreType.DMA,
        pltpu.SemaphoreType.DMA,
        pltpu.SemaphoreType.DMA,
        pltpu.SemaphoreType.DMA,
        pltpu.SemaphoreType.DMA,
        pltpu.SemaphoreType.DMA,
        pltpu.SemaphoreType.DMA,
        pltpu.SemaphoreType.DMA,
        pltpu.SemaphoreType.DMA,
    ],
)


def _tc_body(p0_ref, p1_ref, x_ref, w_ref, o_ref):
  acc = p0_ref[...] + p1_ref[...] + x_ref[...]
  y = jnp.dot(acc, w_ref[...], preferred_element_type=jnp.float32)
  o_ref[...] = jnp.where(y >= 0, y, 0.2 * y)


_BR = 1000


def _tc_proj(p0, p1, X, W):
  return pl.pallas_call(
      _tc_body,
      grid=(N // _BR,),
      in_specs=[
          pl.BlockSpec((_BR, D), lambda i: (i, 0)),
          pl.BlockSpec((_BR, D), lambda i: (i, 0)),
          pl.BlockSpec((_BR, D), lambda i: (i, 0)),
          pl.BlockSpec((D, D), lambda i: (0, 0)),
      ],
      out_specs=pl.BlockSpec((_BR, D), lambda i: (i, 0)),
      out_shape=jax.ShapeDtypeStruct((N, D), jnp.float32),
  )(p0, p1, X, W)


@jax.jit
def kernel(X, edge_values, W, edge_index):
  dst = edge_index[0].reshape(NW, NCHUNK, C)
  src = edge_index[1].reshape(NW, NCHUNK, C)
  ev = edge_values.reshape(NW, NCHUNK, C)
  epacked = jnp.stack([src, dst], axis=2)
  partials = _sc_agg(X, epacked, ev)
  return _tc_proj(partials[0], partials[1], X, W)


# speedup vs baseline: 10.5038x; 1.0559x over previous
import functools

import jax
import jax.numpy as jnp
from jax import lax
from jax.experimental import pallas as pl
from jax.experimental.pallas import tpu as pltpu
from jax.experimental.pallas import tpu_sc as plsc

N = 10000
D = 128
E = 320000
NC = 2
NS = 16
NW = NC * NS
EPW = E // NW
C = 80
NCHUNK = EPW // C
ZR = C
NZCHUNK = N // ZR
ZPS = -(-NZCHUNK // NS)
NTRI = -(-NCHUNK // 3)


def _sc_body(x_hbm, src_hbm, dst_hbm, ev_hbm, out_hbm,
             sb0, sb1, sb2, db0, db1, db2, ev0, ev1, ev2,
             rows0, rows1, rows2, agg_sh,
             isem0, isem1, isem2, gsem0, gsem1, gsem2,
             ssem0, ssem1, ssem2):
  cid = lax.axis_index("c")
  sid = lax.axis_index("s")
  wid = cid * NS + sid
  sb = (sb0, sb1, sb2)
  db = (db0, db1, db2)
  evb = (ev0, ev1, ev2)
  rows = (rows0, rows1, rows2)
  isem = (isem0, isem1, isem2)
  gsem = (gsem0, gsem1, gsem2)
  ssem = (ssem0, ssem1, ssem2)

  for c0 in range(2):
    pltpu.async_copy(src_hbm.at[wid, c0], sb[c0], isem[c0])
    pltpu.async_copy(dst_hbm.at[wid, c0], db[c0], isem[c0])
    pltpu.async_copy(ev_hbm.at[wid, c0], evb[c0], isem[c0])
  pltpu.make_async_copy(src_hbm.at[wid, 0], sb[0], isem[0]).wait()
  pltpu.make_async_copy(dst_hbm.at[wid, 0], db[0], isem[0]).wait()
  pltpu.make_async_copy(ev_hbm.at[wid, 0], evb[0], isem[0]).wait()
  pltpu.async_copy(x_hbm.at[sb[0]], rows[0], gsem[0])

  def zero_row(r, _):
    for dpart in range(D // 16):
      rows2[r, pl.ds(dpart * 16, 16)] = jnp.zeros((16,), jnp.float32)
    return 0
  lax.fori_loop(0, ZR, zero_row, 0)
  for z in range(ZPS):
    t = sid + z * NS
    @pl.when(t < NZCHUNK)
    def _():
      pltpu.sync_copy(rows2, agg_sh.at[pl.ds(t * ZR, ZR)])
  plsc.subcore_barrier()

  def scale_rows(rb, evv):
    def scale_group(g, _):
      ev16 = evv[pl.ds(g * 16, 16)]
      for j in range(16):
        e = g * 16 + j
        evs = lax.gather(
            ev16, jnp.full((16, 1), j, jnp.int32),
            lax.GatherDimensionNumbers(
                offset_dims=(), collapsed_slice_dims=(0,),
                start_index_map=(0,)),
            (1,), mode=lax.GatherScatterMode.PROMISE_IN_BOUNDS)
        for dpart in range(D // 16):
          sl = pl.ds(dpart * 16, 16)
          rb[e, sl] = rb[e, sl] * evs
      return 0
    lax.fori_loop(0, C // 16, scale_group, 0)

  def tri_body(p, _):
    for b in range(3):
      nb = (b + 1) % 3
      pb = (b + 2) % 3
      c = p * 3 + b
      @pl.when(c < NCHUNK)
      def _():
        @pl.when(c >= 1)
        def _():
          pltpu.make_async_copy(
              rows[pb], agg_sh.at[db[pb]], ssem[pb]).wait()
        @pl.when(c + 2 < NCHUNK)
        def _():
          pltpu.async_copy(src_hbm.at[wid, c + 2], sb[pb], isem[pb])
          pltpu.async_copy(dst_hbm.at[wid, c + 2], db[pb], isem[pb])
          pltpu.async_copy(ev_hbm.at[wid, c + 2], evb[pb], isem[pb])
        @pl.when(c + 1 < NCHUNK)
        def _():
          pltpu.make_async_copy(
              src_hbm.at[wid, c + 1], sb[nb], isem[nb]).wait()
          pltpu.make_async_copy(
              dst_hbm.at[wid, c + 1], db[nb], isem[nb]).wait()
          pltpu.make_async_copy(
              ev_hbm.at[wid, c + 1], evb[nb], isem[nb]).wait()
          pltpu.async_copy(x_hbm.at[sb[nb]], rows[nb], gsem[nb])
        pltpu.make_async_copy(
            x_hbm.at[sb[b]], rows[b], gsem[b]).wait()
        scale_rows(rows[b], evb[b])
        pltpu.async_copy(rows[b], agg_sh.at[db[b]], ssem[b],
                         add=True)
    return 0
  lax.fori_loop(0, NTRI, tri_body, 0)
  lastb = (NCHUNK - 1) % 3
  pltpu.make_async_copy(
      rows[lastb], agg_sh.at[db[lastb]], ssem[lastb]).wait()
  plsc.subcore_barrier()

  for z in range(ZPS):
    t = sid + z * NS
    @pl.when(t < NZCHUNK)
    def _():
      pltpu.sync_copy(agg_sh.at[pl.ds(t * ZR, ZR)], rows2)
      pltpu.sync_copy(rows2, out_hbm.at[cid, pl.ds(t * ZR, ZR)])


_sc_agg = pl.kernel(
    _sc_body,
    out_type=jax.ShapeDtypeStruct((NC, N, D), jnp.float32),
    mesh=plsc.VectorSubcoreMesh(
        core_axis_name="c", subcore_axis_name="s",
        num_cores=NC, num_subcores=NS),
    scratch_types=[
        pltpu.VMEM((C,), jnp.int32),
        pltpu.VMEM((C,), jnp.int32),
        pltpu.VMEM((C,), jnp.int32),
        pltpu.VMEM((C,), jnp.int32),
        pltpu.VMEM((C,), jnp.int32),
        pltpu.VMEM((C,), jnp.int32),
        pltpu.VMEM((C,), jnp.float32),
        pltpu.VMEM((C,), jnp.float32),
        pltpu.VMEM((C,), jnp.float32),
        pltpu.VMEM((C, D), jnp.float32),
        pltpu.VMEM((C, D), jnp.float32),
        pltpu.VMEM((C, D), jnp.float32),
        pltpu.VMEM_SHARED((N, D), jnp.float32),
        pltpu.SemaphoreType.DMA,
        pltpu.SemaphoreType.DMA,
        pltpu.SemaphoreType.DMA,
        pltpu.SemaphoreType.DMA,
        pltpu.SemaphoreType.DMA,
        pltpu.SemaphoreType.DMA,
        pltpu.SemaphoreType.DMA,
        pltpu.SemaphoreType.DMA,
        pltpu.SemaphoreType.DMA,
    ],
)


def _tc_body(p0_ref, p1_ref, x_ref, w_ref, o_ref):
  acc = p0_ref[...] + p1_ref[...] + x_ref[...]
  y = jnp.dot(acc, w_ref[...], preferred_element_type=jnp.float32)
  o_ref[...] = jnp.where(y >= 0, y, 0.2 * y)


_BR = 1000


def _tc_proj(p0, p1, X, W):
  return pl.pallas_call(
      _tc_body,
      grid=(N // _BR,),
      in_specs=[
          pl.BlockSpec((_BR, D), lambda i: (i, 0)),
          pl.BlockSpec((_BR, D), lambda i: (i, 0)),
          pl.BlockSpec((_BR, D), lambda i: (i, 0)),
          pl.BlockSpec((D, D), lambda i: (0, 0)),
      ],
      out_specs=pl.BlockSpec((_BR, D), lambda i: (i, 0)),
      out_shape=jax.ShapeDtypeStruct((N, D), jnp.float32),
  )(p0, p1, X, W)


@jax.jit
def kernel(X, edge_values, W, edge_index):
  dst = edge_index[0].reshape(NW, NCHUNK, C)
  src = edge_index[1].reshape(NW, NCHUNK, C)
  ev = edge_values.reshape(NW, NCHUNK, C)
  partials = _sc_agg(X, src, dst, ev)
  return _tc_proj(partials[0], partials[1], X, W)
